# Initial kernel scaffold; baseline (speedup 1.0000x reference)
import jax
import jax.numpy as jnp
from jax.experimental import pallas as pl


def kernel(xyz, W1, b1, W2, b2):
    raise NotImplementedError("write your pallas kernel here")


# speedup vs baseline: 18.2275x; 18.2275x over previous
import jax
import jax.numpy as jnp
from jax import lax
from jax.experimental import pallas as pl

K = 16
R = 128


def _body(x_ref, xt_ref, w1_ref, b1_ref, w2_ref, b2_ref, out_ref):
    Rr, N = out_ref.shape[2], out_ref.shape[3]
    H = out_ref.shape[1]

    x = x_ref[0]
    xt = xt_ref[0]

    sq_row = x[0:1, :] * x[0:1, :] + x[1:2, :] * x[1:2, :] + x[2:3, :] * x[2:3, :]
    sq_col = jnp.sum(xt * xt, axis=1, keepdims=True)
    dot = jnp.dot(xt, x, preferred_element_type=jnp.float32)
    acc = sq_col + sq_row - 2.0 * dot

    jiota = lax.broadcasted_iota(jnp.int32, (Rr, N), 1)
    w1 = w1_ref[...]
    b1 = b1_ref[...]
    w2 = w2_ref[...]
    b2 = b2_ref[...]

    for h in range(H):
        out_ref[0, h, :, :] = jnp.zeros((Rr, N), jnp.float32)

    for _ in range(K):
        m = jnp.min(acc, axis=1, keepdims=True)
        idxk = jnp.min(jnp.where(acc == m, jiota, N), axis=1, keepdims=True)
        maskb = jiota == idxk
        mask = maskb.astype(jnp.float32)
        acc = jnp.where(maskb, jnp.inf, acc)

        nx = jnp.sum(mask * x[0:1, :], axis=1, keepdims=True)
        ny = jnp.sum(mask * x[1:2, :], axis=1, keepdims=True)
        nz = jnp.sum(mask * x[2:3, :], axis=1, keepdims=True)
        relx = xt[:, 0:1] - nx
        rely = xt[:, 1:2] - ny
        relz = xt[:, 2:3] - nz

        hid = jnp.maximum(
            relx * w1[0:1, :] + rely * w1[1:2, :] + relz * w1[2:3, :] + b1, 0.0
        )
        biask = jnp.dot(hid, w2, preferred_element_type=jnp.float32) + b2

        for h in range(H):
            out_ref[0, h, :, :] += mask * biask[:, h:h + 1]


def kernel(xyz, W1, b1, W2, b2):
    B, _, N = xyz.shape
    HID = W1.shape[1]
    H = W2.shape[1]
    xt = jnp.transpose(xyz, (0, 2, 1))
    b1r = b1.reshape(1, HID)
    b2r = b2.reshape(1, H)

    return pl.pallas_call(
        _body,
        grid=(B, N // R),
        in_specs=[
            pl.BlockSpec((1, 3, N), lambda b, j: (b, 0, 0)),
            pl.BlockSpec((1, R, 3), lambda b, j: (b, j, 0)),
            pl.BlockSpec((3, HID), lambda b, j: (0, 0)),
            pl.BlockSpec((1, HID), lambda b, j: (0, 0)),
            pl.BlockSpec((HID, H), lambda b, j: (0, 0)),
            pl.BlockSpec((1, H), lambda b, j: (0, 0)),
        ],
        out_specs=pl.BlockSpec((1, H, R, N), lambda b, j: (b, 0, j, 0)),
        out_shape=jax.ShapeDtypeStruct((B, H, N, N), jnp.float32),
    )(xyz, xt, W1, b1r, W2, b2r)
